# baseline (speedup 1.0000x reference)
import functools

import jax
import jax.numpy as jnp
from jax import lax
from jax.experimental import pallas as pl
from jax.experimental.pallas import tpu as pltpu
from jax.experimental.pallas import tpu_sc as plsc

N = 50000
E = 100000
C = 5
NF = 64
SF = 16
P = 3

CK = 128
E_PAD = 102400
W128 = 128
FW = C * SF
NR = 6
QR = 8352
QACC = QR + CK
TRASHQ = QR
ZQ = 96
NZCH = QR // ZQ
LASTQ = NR - 1
PER_T = (E_PAD // CK) // 16
NCH = P * (E_PAD // CK)
WCH = NCH // 32
WELT = WCH * CK
KB = 50
NB = (NCH // 16) // KB

F32 = jnp.float32
INTERPRET = False


def _k1_body(m_ref, w_ref, t_ref, a_ref):
    x = m_ref[0]
    bn = x.shape[0]
    tc, ac = [], []
    for c in range(C):
        y = jnp.dot(x[:, c, :], w_ref[0, c], preferred_element_type=F32)
        tc.append(y[:, :SF])
        ac.append(y[:, SF:])
    z48 = jnp.zeros((bn, W128 - FW), dtype=F32)
    t_ref[0] = jnp.concatenate(tc + [z48], axis=1)
    a_ref[0] = jnp.concatenate(ac + [z48], axis=1)


BN = 2000


def _k1(ms, wk1):
    return pl.pallas_call(
        _k1_body,
        grid=(P, N // BN),
        in_specs=[
            pl.BlockSpec((1, BN, C, NF), lambda p, i: (p, i, 0, 0)),
            pl.BlockSpec((1, C, NF, 2 * SF), lambda p, i: (p, 0, 0, 0)),
        ],
        out_specs=[
            pl.BlockSpec((1, BN, W128), lambda p, i: (p, i, 0)),
            pl.BlockSpec((1, BN, W128), lambda p, i: (p, i, 0)),
        ],
        out_shape=[
            jax.ShapeDtypeStruct((P, N, W128), F32),
            jax.ShapeDtypeStruct((P, N, W128), F32),
        ],
        interpret=INTERPRET,
    )(ms, wk1)


def _k3_body(z_ref, b3a_ref, w3b_ref, b3b_ref, tab_ref):
    bn = z_ref.shape[0]
    h = jnp.tanh(z_ref[:, :FW] + b3a_ref[...])
    msp = []
    for c in range(C):
        hc = h[:, c * SF:(c + 1) * SF]
        msp.append(jnp.tanh(
            jnp.dot(hc, w3b_ref[c], preferred_element_type=F32)
            + b3b_ref[0, c * SF:(c + 1) * SF]))
    z48 = jnp.zeros((bn, W128 - FW), dtype=F32)
    tab_ref[...] = jnp.concatenate(msp + [z48], axis=1)


def _k3(z, b3a_flat, w3b, b3b_flat):
    return pl.pallas_call(
        _k3_body,
        grid=(N // BN,),
        in_specs=[
            pl.BlockSpec((BN, W128), lambda i: (i, 0)),
            pl.BlockSpec((1, FW), lambda i: (0, 0)),
            pl.BlockSpec((C, SF, SF), lambda i: (0, 0, 0)),
            pl.BlockSpec((1, FW), lambda i: (0, 0)),
        ],
        out_specs=pl.BlockSpec((BN, W128), lambda i: (i, 0)),
        out_shape=jax.ShapeDtypeStruct((N, W128), F32),
        interpret=INTERPRET,
    )(z, b3a_flat, w3b, b3b_flat)


BE = 2048


def _k5_body(ga_ref, gm_ref, wbot_ref, be1_ref, w2_ref, be2_ref, ebd_ref,
             msg_ref):
    ga = ga_ref[0]
    gm = gm_ref[0]
    msp = gm[:, :FW]
    bs = []
    for c in range(C):
        bs.append(jnp.dot(msp[:, c * SF:(c + 1) * SF], wbot_ref[0, c],
                          preferred_element_type=F32)
                  + be1_ref[0, 0, c * SF:(c + 1) * SF])
    e1 = jnp.tanh(ga[:, :FW] + jnp.concatenate(bs, axis=1))
    logits = jnp.dot(e1, w2_ref[0], preferred_element_type=F32) + be2_ref[0]
    mx = jnp.max(logits, axis=1, keepdims=True)
    ex = jnp.exp(logits - mx)
    att = ex / jnp.sum(ex, axis=1, keepdims=True)
    expand = jnp.dot(att, ebd_ref[...], preferred_element_type=F32)
    msg80 = expand * msp
    one = jnp.ones((ga.shape[0], 1), dtype=F32)
    z47 = jnp.zeros((ga.shape[0], W128 - FW - 1), dtype=F32)
    msg_ref[0] = jnp.concatenate([msg80, one, z47], axis=1)


def _k5(ga, gm, wbot, be1, w2bd, be2, ebd):
    return pl.pallas_call(
        _k5_body,
        grid=(P, E_PAD // BE),
        in_specs=[
            pl.BlockSpec((1, BE, W128), lambda p, i: (p, i, 0)),
            pl.BlockSpec((1, BE, W128), lambda p, i: (p, i, 0)),
            pl.BlockSpec((1, C, SF, SF), lambda p, i: (p, 0, 0, 0)),
            pl.BlockSpec((1, 1, FW), lambda p, i: (p, 0, 0)),
            pl.BlockSpec((1, FW, C), lambda p, i: (p, 0, 0)),
            pl.BlockSpec((1, 1, C), lambda p, i: (p, 0, 0)),
            pl.BlockSpec((C, FW), lambda p, i: (0, 0)),
        ],
        out_specs=pl.BlockSpec((1, BE, W128), lambda p, i: (p, i, 0)),
        out_shape=jax.ShapeDtypeStruct((P, E_PAD, W128), F32),
        interpret=INTERPRET,
    )(ga, gm, wbot, be1, w2bd, be2, ebd)


def _k7_body(m_ref, ss_ref, wn1_ref, bn1_ref, wn2_ref, bn2_ref, o_ref):
    m = m_ref[0]
    ss = ss_ref[0]
    cnt = jnp.clip(ss[:, FW:FW + 1], 1.0, None)
    outs = []
    for c in range(C):
        mean_c = ss[:, c * SF:(c + 1) * SF] / cnt
        mcat = jnp.concatenate([m[:, c, :], mean_c], axis=1)
        h = jnp.tanh(jnp.dot(mcat, wn1_ref[0, c], preferred_element_type=F32)
                     + bn1_ref[0, c])
        h = jnp.tanh(jnp.dot(h, wn2_ref[0, c], preferred_element_type=F32)
                     + bn2_ref[0, c])
        outs.append(h[:, None, :])
    o_ref[0] = jnp.concatenate(outs, axis=1)


def _k7(ms, ssum, wn1, bn1, wn2, bn2):
    return pl.pallas_call(
        _k7_body,
        grid=(P, N // BN),
        in_specs=[
            pl.BlockSpec((1, BN, C, NF), lambda p, i: (p, i, 0, 0)),
            pl.BlockSpec((1, BN, W128), lambda p, i: (p, i, 0)),
            pl.BlockSpec((1, C, NF + SF, NF), lambda p, i: (p, 0, 0, 0)),
            pl.BlockSpec((1, C, NF), lambda p, i: (p, 0, 0)),
            pl.BlockSpec((1, C, NF, NF), lambda p, i: (p, 0, 0, 0)),
            pl.BlockSpec((1, C, NF), lambda p, i: (p, 0, 0)),
        ],
        out_specs=pl.BlockSpec((1, BN, C, NF), lambda p, i: (p, i, 0, 0)),
        out_shape=jax.ShapeDtypeStruct((P, N, C, NF), F32),
        interpret=INTERPRET,
    )(ms, ssum, wn1, bn1, wn2, bn2)


def _sc_mesh():
    return plsc.VectorSubcoreMesh(core_axis_name="c", subcore_axis_name="s",
                                  num_cores=2, num_subcores=16)


def _zero_acc(zeros_ref, acc, sid):
    def zbody(k, _):
        j = sid + 16 * k

        @pl.when(j < NZCH)
        def _():
            pltpu.sync_copy(zeros_ref.at[pl.ds(0, ZQ)],
                            acc.at[pl.ds(j * ZQ, ZQ)])
        return 0

    lax.fori_loop(0, (NZCH + 15) // 16, zbody, 0)

    @pl.when(sid == 0)
    def _():
        pltpu.sync_copy(zeros_ref.at[pl.ds(0, CK)],
                        acc.at[pl.ds(QR, CK)])


def _copy_out(acc, out_hbm, sid, q, out_base):
    def obody(k, _):
        j = sid + 16 * k
        full = (j < NZCH) & ((q < LASTQ) | (j < 85))
        tail = (q == LASTQ) & (j == 85)

        @pl.when(full)
        def _():
            pltpu.sync_copy(acc.at[pl.ds(j * ZQ, ZQ)],
                            out_hbm.at[pl.ds(out_base + j * ZQ, ZQ)])

        @pl.when(tail)
        def _():
            pltpu.sync_copy(acc.at[pl.ds(85 * ZQ, 80)],
                            out_hbm.at[pl.ds(out_base + 85 * ZQ, 80)])
        return 0

    lax.fori_loop(0, (NZCH + 15) // 16, obody, 0)


def _k2(t_flat, hits_flat, sps_flat, zeros_hbm):
    @functools.partial(
        pl.kernel,
        out_type=jax.ShapeDtypeStruct((N, W128), F32),
        mesh=_sc_mesh(),
        scratch_types=[
            pltpu.VMEM((KB * CK,), jnp.int32),
            pltpu.VMEM((KB * CK,), jnp.int32),
            pltpu.VMEM((CK,), jnp.int32),
            pltpu.VMEM((CK,), jnp.int32),
            pltpu.VMEM((CK,), jnp.int32),
            pltpu.VMEM((CK,), jnp.int32),
            pltpu.VMEM((CK, W128), F32),
            pltpu.VMEM((CK, W128), F32),
            pltpu.VMEM_SHARED((QACC, W128), F32),
            pltpu.SemaphoreType.DMA,
            pltpu.SemaphoreType.DMA,
            pltpu.SemaphoreType.DMA,
            pltpu.SemaphoreType.DMA,
        ],
    )
    def k2(t_hbm, hits_hbm, sps_hbm, zeros_ref, z_hbm,
           hraw, sraw, gidx0, gidx1, sidx0, sidx1, rows0, rows1, acc,
           sg0, sg1, ss0, ss1):
        cid = lax.axis_index("c")
        sid = lax.axis_index("s")
        lane = jnp.arange(16, dtype=jnp.int32)
        tb = sid * (NCH // 16)

        def prep(bb, c_local, base, gidx_v, sidx_v):
            pn = (((tb + bb + c_local) * CK) // E_PAD) * N
            for r in range(CK // 16):
                h16 = hraw[pl.ds(c_local * CK + r * 16, 16)]
                s16 = sraw[pl.ds(c_local * CK + r * 16, 16)]
                roff = r * 16 + lane
                l16 = s16 - base
                ok = (l16 >= 0) & (l16 < QR)
                sidx_v[pl.ds(r * 16, 16)] = jnp.where(ok, l16, TRASHQ + roff)
                g16 = jnp.minimum(h16, N - 1) + pn
                gidx_v[pl.ds(r * 16, 16)] = jnp.where(ok, g16, pn + roff)

        for qq in range(NR // 2):
            q = cid * (NR // 2) + qq
            base = q * QR
            _zero_acc(zeros_ref, acc, sid)
            plsc.subcore_barrier()

            for b in range(NB):
                bb = b * KB
                eb = (tb + bb) * CK
                pltpu.sync_copy(hits_hbm.at[pl.ds(eb, KB * CK)], hraw)
                pltpu.sync_copy(sps_hbm.at[pl.ds(eb, KB * CK)], sraw)

                def pair(kk, _):
                    a = 2 * kk
                    prep(bb, a, base, gidx0, sidx0)
                    cg0 = pltpu.async_copy(t_hbm.at[gidx0], rows0, sg0)
                    prep(bb, a + 1, base, gidx1, sidx1)
                    cg1 = pltpu.async_copy(t_hbm.at[gidx1], rows1, sg1)
                    cg0.wait()
                    cs0 = pltpu.async_copy(rows0, acc.at[sidx0], ss0,
                                           add=True)
                    cg1.wait()
                    cs1 = pltpu.async_copy(rows1, acc.at[sidx1], ss1,
                                           add=True)
                    cs0.wait()
                    cs1.wait()
                    return 0

                lax.fori_loop(0, KB // 2, pair, 0)
            plsc.subcore_barrier()
            _copy_out(acc, z_hbm, sid, q, base)
            plsc.subcore_barrier()

    return k2(t_flat, hits_flat, sps_flat, zeros_hbm)


def _k4(a_flat, msp_tab, hits_flat, sps_flat):
    @functools.partial(
        pl.kernel,
        out_type=[
            jax.ShapeDtypeStruct((P * E_PAD, W128), F32),
            jax.ShapeDtypeStruct((P * E_PAD, W128), F32),
        ],
        mesh=_sc_mesh(),
        scratch_types=[
            pltpu.VMEM((WELT,), jnp.int32),
            pltpu.VMEM((WELT,), jnp.int32),
            pltpu.VMEM((CK, W128), F32),
            pltpu.VMEM((CK, W128), F32),
            pltpu.VMEM((CK, W128), F32),
            pltpu.VMEM((CK, W128), F32),
            pltpu.SemaphoreType.DMA,
            pltpu.SemaphoreType.DMA,
            pltpu.SemaphoreType.DMA,
            pltpu.SemaphoreType.DMA,
            pltpu.SemaphoreType.DMA,
            pltpu.SemaphoreType.DMA,
            pltpu.SemaphoreType.DMA,
            pltpu.SemaphoreType.DMA,
        ],
    )
    def k4(a_hbm, msp_hbm, hits_hbm, sps_hbm, ga_hbm, gm_hbm,
           gh_v, gs_v, ra0, ra1, rm0, rm1,
           sga0, sga1, sgm0, sgm1, swa0, swa1, swm0, swm1):
        cid = lax.axis_index("c")
        sid = lax.axis_index("s")
        w = sid * 2 + cid
        eb = w * WELT

        pltpu.sync_copy(hits_hbm.at[pl.ds(eb, WELT)], gh_v)
        pltpu.sync_copy(sps_hbm.at[pl.ds(eb, WELT)], gs_v)

        def xform(r, _):
            pnum = (eb + r * 16) // E_PAD
            h16 = gh_v[pl.ds(r * 16, 16)]
            s16 = gs_v[pl.ds(r * 16, 16)]
            gh_v[pl.ds(r * 16, 16)] = jnp.minimum(h16, N - 1) + pnum * N
            gs_v[pl.ds(r * 16, 16)] = jnp.minimum(s16, N - 1)
            return 0

        lax.fori_loop(0, WELT // 16, xform, 0)

        def pair(kk, _):
            a = 2 * kk
            b = a + 1
            offa = eb + a * CK
            offb = eb + b * CK
            ca0 = pltpu.async_copy(
                a_hbm.at[gh_v.at[pl.ds(a * CK, CK)]], ra0, sga0)
            cm0 = pltpu.async_copy(
                msp_hbm.at[gs_v.at[pl.ds(a * CK, CK)]], rm0, sgm0)
            ca1 = pltpu.async_copy(
                a_hbm.at[gh_v.at[pl.ds(b * CK, CK)]], ra1, sga1)
            cm1 = pltpu.async_copy(
                msp_hbm.at[gs_v.at[pl.ds(b * CK, CK)]], rm1, sgm1)
            ca0.wait()
            cm0.wait()
            wa0 = pltpu.async_copy(ra0, ga_hbm.at[pl.ds(offa, CK)], swa0)
            wm0 = pltpu.async_copy(rm0, gm_hbm.at[pl.ds(offa, CK)], swm0)
            ca1.wait()
            cm1.wait()
            wa1 = pltpu.async_copy(ra1, ga_hbm.at[pl.ds(offb, CK)], swa1)
            wm1 = pltpu.async_copy(rm1, gm_hbm.at[pl.ds(offb, CK)], swm1)
            wa0.wait()
            wm0.wait()
            wa1.wait()
            wm1.wait()
            return 0

        lax.fori_loop(0, WCH // 2, pair, 0)

        t = WCH - 1
        offt = eb + t * CK
        ca0 = pltpu.async_copy(
            a_hbm.at[gh_v.at[pl.ds(t * CK, CK)]], ra0, sga0)
        cm0 = pltpu.async_copy(
            msp_hbm.at[gs_v.at[pl.ds(t * CK, CK)]], rm0, sgm0)
        ca0.wait()
        cm0.wait()
        pltpu.sync_copy(ra0, ga_hbm.at[pl.ds(offt, CK)])
        pltpu.sync_copy(rm0, gm_hbm.at[pl.ds(offt, CK)])

    return k4(a_flat, msp_tab, hits_flat, sps_flat)


def _k6(msg_flat, hits_flat, zeros_hbm):
    @functools.partial(
        pl.kernel,
        out_type=jax.ShapeDtypeStruct((P * N, W128), F32),
        mesh=_sc_mesh(),
        scratch_types=[
            pltpu.VMEM((PER_T * CK,), jnp.int32),
            pltpu.VMEM((CK,), jnp.int32),
            pltpu.VMEM((CK,), jnp.int32),
            pltpu.VMEM((CK,), jnp.int32),
            pltpu.VMEM((CK,), jnp.int32),
            pltpu.VMEM((CK, W128), F32),
            pltpu.VMEM((CK, W128), F32),
            pltpu.VMEM_SHARED((QACC, W128), F32),
            pltpu.SemaphoreType.DMA,
            pltpu.SemaphoreType.DMA,
            pltpu.SemaphoreType.DMA,
            pltpu.SemaphoreType.DMA,
        ],
    )
    def k6(msg_hbm, hits_hbm, zeros_ref, ss_hbm,
           hraw, eidx0, eidx1, sidx0, sidx1, rows0, rows1, acc,
           sg0, sg1, ss0, ss1):
        cid = lax.axis_index("c")
        sid = lax.axis_index("s")
        lane = jnp.arange(16, dtype=jnp.int32)
        pch = PER_T

        for qq in range(NR // 2):
            q = cid * (NR // 2) + qq
            base = q * QR
            for p in range(P):
                eb = p * E_PAD + sid * (pch * CK)
                pltpu.sync_copy(hits_hbm.at[pl.ds(eb, pch * CK)], hraw)
                _zero_acc(zeros_ref, acc, sid)
                plsc.subcore_barrier()

                def prep(c_local, eidx_v, sidx_v):
                    for r in range(CK // 16):
                        h16 = hraw[pl.ds(c_local * CK + r * 16, 16)]
                        roff = r * 16 + lane
                        l16 = h16 - base
                        ok = (l16 >= 0) & (l16 < QR)
                        sidx_v[pl.ds(r * 16, 16)] = jnp.where(
                            ok, l16, TRASHQ + roff)
                        e16 = sid * (pch * CK) + c_local * CK + roff
                        eidx_v[pl.ds(r * 16, 16)] = (
                            jnp.where(ok, e16, roff) + p * E_PAD)

                def pair(kk, _):
                    a = 2 * kk
                    b = a + 1
                    prep(a, eidx0, sidx0)
                    cg0 = pltpu.async_copy(msg_hbm.at[eidx0], rows0, sg0)
                    prep(b, eidx1, sidx1)
                    cg1 = pltpu.async_copy(msg_hbm.at[eidx1], rows1, sg1)
                    cg0.wait()
                    cs0 = pltpu.async_copy(rows0, acc.at[sidx0], ss0,
                                           add=True)
                    cg1.wait()
                    cs1 = pltpu.async_copy(rows1, acc.at[sidx1], ss1,
                                           add=True)
                    cs0.wait()
                    cs1.wait()
                    return 0

                lax.fori_loop(0, pch // 2, pair, 0)
                plsc.subcore_barrier()
                _copy_out(acc, ss_hbm, sid, q, p * N + base)
                plsc.subcore_barrier()

    return k6(msg_flat, hits_flat, zeros_hbm)


def kernel(m_u, m_v, m_y, edge_index_u, edge_index_v, edge_index_y,
           W3a, b3a, W3b, b3b, We1, be1, We2, be2, Wn1, bn1, Wn2, bn2):
    ms = jnp.stack([m_u, m_v, m_y])
    hits = jnp.stack([edge_index_u[0], edge_index_v[0], edge_index_y[0]])
    sps = jnp.stack([edge_index_u[1], edge_index_v[1], edge_index_y[1]])
    hits = hits.astype(jnp.int32)
    sps = sps.astype(jnp.int32)
    pad = jnp.full((P, E_PAD - E), N, dtype=jnp.int32)
    hits_flat = jnp.concatenate([hits, pad], axis=1).reshape(P * E_PAD)
    sps_flat = jnp.concatenate([sps, pad], axis=1).reshape(P * E_PAD)
    zeros_hbm = jnp.zeros((CK, W128), dtype=F32)

    w3a_pl = W3a.reshape(C, P, NF, SF).transpose(1, 0, 2, 3)
    we1_top = We1[:, :, :NF, :]
    we1_bot = We1[:, :, NF:, :]
    wk1 = jnp.concatenate([w3a_pl, we1_top], axis=-1)
    b3a_flat = b3a.reshape(1, FW)
    b3b_flat = b3b.reshape(1, FW)
    be1_flat = be1.reshape(P, 1, FW)
    w2bd = (We2[:, :, :, 0][:, :, :, None]
            * jnp.eye(C, dtype=F32)[None, :, None, :]).reshape(P, FW, C)
    be2_flat = be2.reshape(P, 1, C)
    ebd = jnp.repeat(jnp.eye(C, dtype=F32), SF, axis=1)

    t_all, a_all = _k1(ms, wk1)

    z = _k2(t_all.reshape(P * N, W128), hits_flat, sps_flat, zeros_hbm)

    msp_tab = _k3(z, b3a_flat, W3b, b3b_flat)

    ga, gm = _k4(a_all.reshape(P * N, W128), msp_tab, hits_flat, sps_flat)

    msg = _k5(ga.reshape(P, E_PAD, W128), gm.reshape(P, E_PAD, W128),
              we1_bot, be1_flat, w2bd, be2_flat, ebd)

    ssum = _k6(msg.reshape(P * E_PAD, W128), hits_flat,
               zeros_hbm).reshape(P, N, W128)

    return _k7(ms, ssum, Wn1, bn1, Wn2, bn2)


# speedup vs baseline: 1.0925x; 1.0925x over previous
import functools

import jax
import jax.numpy as jnp
from jax import lax
from jax.experimental import pallas as pl
from jax.experimental.pallas import tpu as pltpu
from jax.experimental.pallas import tpu_sc as plsc

N = 50000
E = 100000
C = 5
NF = 64
SF = 16
P = 3

CK = 128
E_PAD = 102400
W128 = 128
FW = C * SF
N_PAD = 50048
QR = 12512
QACC = QR + CK
TRASHQ = QR
ZQ = 544
NZCH = QR // ZQ
PER_T = (E_PAD // CK) // 16
NCH = P * (E_PAD // CK)
WCH = NCH // 32
WELT = WCH * CK

F32 = jnp.float32
INTERPRET = False


def _k1_body(m_ref, w_ref, t_ref, a_ref):
    x = m_ref[0]
    bn = x.shape[0]
    tc, ac = [], []
    for c in range(C):
        y = jnp.dot(x[:, c, :], w_ref[0, c], preferred_element_type=F32)
        tc.append(y[:, :SF])
        ac.append(y[:, SF:])
    z48 = jnp.zeros((bn, W128 - FW), dtype=F32)
    t_ref[0] = jnp.concatenate(tc + [z48], axis=1)
    a_ref[0] = jnp.concatenate(ac + [z48], axis=1)


BN = 2000


def _k1(ms, wk1):
    return pl.pallas_call(
        _k1_body,
        grid=(P, N // BN),
        in_specs=[
            pl.BlockSpec((1, BN, C, NF), lambda p, i: (p, i, 0, 0)),
            pl.BlockSpec((1, C, NF, 2 * SF), lambda p, i: (p, 0, 0, 0)),
        ],
        out_specs=[
            pl.BlockSpec((1, BN, W128), lambda p, i: (p, i, 0)),
            pl.BlockSpec((1, BN, W128), lambda p, i: (p, i, 0)),
        ],
        out_shape=[
            jax.ShapeDtypeStruct((P, N, W128), F32),
            jax.ShapeDtypeStruct((P, N, W128), F32),
        ],
        interpret=INTERPRET,
    )(ms, wk1)


def _k3_body(z_ref, b3a_ref, w3b_ref, b3b_ref, tab_ref):
    bn = z_ref.shape[0]
    h = jnp.tanh(z_ref[:, :FW] + b3a_ref[...])
    msp = []
    for c in range(C):
        hc = h[:, c * SF:(c + 1) * SF]
        msp.append(jnp.tanh(
            jnp.dot(hc, w3b_ref[c], preferred_element_type=F32)
            + b3b_ref[0, c * SF:(c + 1) * SF]))
    z48 = jnp.zeros((bn, W128 - FW), dtype=F32)
    tab_ref[...] = jnp.concatenate(msp + [z48], axis=1)


def _k3(z, b3a_flat, w3b, b3b_flat):
    return pl.pallas_call(
        _k3_body,
        grid=(N // BN,),
        in_specs=[
            pl.BlockSpec((BN, W128), lambda i: (i, 0)),
            pl.BlockSpec((1, FW), lambda i: (0, 0)),
            pl.BlockSpec((C, SF, SF), lambda i: (0, 0, 0)),
            pl.BlockSpec((1, FW), lambda i: (0, 0)),
        ],
        out_specs=pl.BlockSpec((BN, W128), lambda i: (i, 0)),
        out_shape=jax.ShapeDtypeStruct((N, W128), F32),
        interpret=INTERPRET,
    )(z, b3a_flat, w3b, b3b_flat)


BE = 4096


def _k5_body(ga_ref, gm_ref, wbot_ref, be1_ref, w2_ref, be2_ref, ebd_ref,
             msg_ref):
    ga = ga_ref[0]
    gm = gm_ref[0]
    msp = gm[:, :FW]
    bs = []
    for c in range(C):
        bs.append(jnp.dot(msp[:, c * SF:(c + 1) * SF], wbot_ref[0, c],
                          preferred_element_type=F32)
                  + be1_ref[0, 0, c * SF:(c + 1) * SF])
    e1 = jnp.tanh(ga[:, :FW] + jnp.concatenate(bs, axis=1))
    logits = jnp.dot(e1, w2_ref[0], preferred_element_type=F32) + be2_ref[0]
    mx = jnp.max(logits, axis=1, keepdims=True)
    ex = jnp.exp(logits - mx)
    att = ex / jnp.sum(ex, axis=1, keepdims=True)
    expand = jnp.dot(att, ebd_ref[...], preferred_element_type=F32)
    msg80 = expand * msp
    one = jnp.ones((ga.shape[0], 1), dtype=F32)
    z47 = jnp.zeros((ga.shape[0], W128 - FW - 1), dtype=F32)
    msg_ref[0] = jnp.concatenate([msg80, one, z47], axis=1)


def _k5(ga, gm, wbot, be1, w2bd, be2, ebd):
    return pl.pallas_call(
        _k5_body,
        grid=(P, E_PAD // BE),
        in_specs=[
            pl.BlockSpec((1, BE, W128), lambda p, i: (p, i, 0)),
            pl.BlockSpec((1, BE, W128), lambda p, i: (p, i, 0)),
            pl.BlockSpec((1, C, SF, SF), lambda p, i: (p, 0, 0, 0)),
            pl.BlockSpec((1, 1, FW), lambda p, i: (p, 0, 0)),
            pl.BlockSpec((1, FW, C), lambda p, i: (p, 0, 0)),
            pl.BlockSpec((1, 1, C), lambda p, i: (p, 0, 0)),
            pl.BlockSpec((C, FW), lambda p, i: (0, 0)),
        ],
        out_specs=pl.BlockSpec((1, BE, W128), lambda p, i: (p, i, 0)),
        out_shape=jax.ShapeDtypeStruct((P, E_PAD, W128), F32),
        interpret=INTERPRET,
    )(ga, gm, wbot, be1, w2bd, be2, ebd)


def _k7_body(m_ref, ss_ref, wn1_ref, bn1_ref, wn2_ref, bn2_ref, o_ref):
    m = m_ref[0]
    ss = ss_ref[0]
    cnt = jnp.clip(ss[:, FW:FW + 1], 1.0, None)
    outs = []
    for c in range(C):
        mean_c = ss[:, c * SF:(c + 1) * SF] / cnt
        mcat = jnp.concatenate([m[:, c, :], mean_c], axis=1)
        h = jnp.tanh(jnp.dot(mcat, wn1_ref[0, c], preferred_element_type=F32)
                     + bn1_ref[0, c])
        h = jnp.tanh(jnp.dot(h, wn2_ref[0, c], preferred_element_type=F32)
                     + bn2_ref[0, c])
        outs.append(h[:, None, :])
    o_ref[0] = jnp.concatenate(outs, axis=1)


def _k7(ms, ssum, wn1, bn1, wn2, bn2):
    return pl.pallas_call(
        _k7_body,
        grid=(P, N // BN),
        in_specs=[
            pl.BlockSpec((1, BN, C, NF), lambda p, i: (p, i, 0, 0)),
            pl.BlockSpec((1, BN, W128), lambda p, i: (p, i, 0)),
            pl.BlockSpec((1, C, NF + SF, NF), lambda p, i: (p, 0, 0, 0)),
            pl.BlockSpec((1, C, NF), lambda p, i: (p, 0, 0)),
            pl.BlockSpec((1, C, NF, NF), lambda p, i: (p, 0, 0, 0)),
            pl.BlockSpec((1, C, NF), lambda p, i: (p, 0, 0)),
        ],
        out_specs=pl.BlockSpec((1, BN, C, NF), lambda p, i: (p, i, 0, 0)),
        out_shape=jax.ShapeDtypeStruct((P, N, C, NF), F32),
        interpret=INTERPRET,
    )(ms, ssum, wn1, bn1, wn2, bn2)


def _sc_mesh():
    return plsc.VectorSubcoreMesh(core_axis_name="c", subcore_axis_name="s",
                                  num_cores=2, num_subcores=16)


def _zero_acc(zeros_ref, acc, sid):
    def zbody(k, _):
        j = sid + 16 * k

        @pl.when(j < NZCH)
        def _():
            pltpu.sync_copy(zeros_ref, acc.at[pl.ds(j * ZQ, ZQ)])
        return 0

    lax.fori_loop(0, (NZCH + 15) // 16, zbody, 0)

    @pl.when(sid == 0)
    def _():
        pltpu.sync_copy(zeros_ref.at[pl.ds(0, CK)],
                        acc.at[pl.ds(QR, CK)])


def _copy_out(acc, out_hbm, sid, q, out_base):
    def obody(k, _):
        j = sid + 16 * k
        last = (q == 3) & (j == NZCH - 1)

        @pl.when((j < NZCH) & jnp.logical_not(last))
        def _():
            pltpu.sync_copy(acc.at[pl.ds(j * ZQ, ZQ)],
                            out_hbm.at[pl.ds(out_base + j * ZQ, ZQ)])

        @pl.when(last)
        def _():
            pltpu.sync_copy(acc.at[pl.ds((NZCH - 1) * ZQ, ZQ - 48)],
                            out_hbm.at[pl.ds(out_base + (NZCH - 1) * ZQ,
                                             ZQ - 48)])
        return 0

    lax.fori_loop(0, (NZCH + 15) // 16, obody, 0)


def _k2(t_flat, hits_flat, sps_flat, zeros_hbm):
    @functools.partial(
        pl.kernel,
        out_type=jax.ShapeDtypeStruct((N_PAD, W128), F32),
        mesh=_sc_mesh(),
        scratch_types=[
            pltpu.VMEM((CK,), jnp.int32),
            pltpu.VMEM((CK,), jnp.int32),
            pltpu.VMEM((CK,), jnp.int32),
            pltpu.VMEM((CK,), jnp.int32),
            pltpu.VMEM((CK, W128), F32),
            pltpu.VMEM_SHARED((QACC, W128), F32),
            pltpu.SemaphoreType.DMA,
        ],
    )
    def k2(t_hbm, hits_hbm, sps_hbm, zeros_ref, z_hbm,
           hit_v, sp_v, gidx_v, sidx_v, rows_v, acc, sem):
        cid = lax.axis_index("c")
        sid = lax.axis_index("s")
        lane = jnp.arange(16, dtype=jnp.int32)

        for qq in range(2):
            q = cid * 2 + qq
            base = q * QR
            _zero_acc(zeros_ref, acc, sid)
            plsc.subcore_barrier()

            for p in range(P):
                def ebody(k, _):
                    j = sid + 16 * k
                    off = p * E_PAD + j * CK
                    pltpu.sync_copy(hits_hbm.at[pl.ds(off, CK)], hit_v)
                    pltpu.sync_copy(sps_hbm.at[pl.ds(off, CK)], sp_v)
                    for r in range(CK // 16):
                        h16 = hit_v[pl.ds(r * 16, 16)]
                        s16 = sp_v[pl.ds(r * 16, 16)]
                        roff = r * 16 + lane
                        l16 = s16 - base
                        ok = (l16 >= 0) & (l16 < QR)
                        sidx_v[pl.ds(r * 16, 16)] = jnp.where(
                            ok, l16, TRASHQ + roff)
                        g16 = jnp.minimum(h16, N - 1) + p * N
                        gidx_v[pl.ds(r * 16, 16)] = jnp.where(
                            ok, g16, p * N + roff)
                    pltpu.async_copy(t_hbm.at[gidx_v], rows_v, sem).wait()
                    pltpu.sync_copy(rows_v, acc.at[sidx_v], add=True)
                    return 0

                lax.fori_loop(0, PER_T, ebody, 0)
            plsc.subcore_barrier()
            _copy_out(acc, z_hbm, sid, q, base)
            plsc.subcore_barrier()

    return k2(t_flat, hits_flat, sps_flat, zeros_hbm)


def _k4(a_flat, msp_tab, hits_flat, sps_flat):
    @functools.partial(
        pl.kernel,
        out_type=[
            jax.ShapeDtypeStruct((P * E_PAD, W128), F32),
            jax.ShapeDtypeStruct((P * E_PAD, W128), F32),
        ],
        mesh=_sc_mesh(),
        scratch_types=[
            pltpu.VMEM((WELT,), jnp.int32),
            pltpu.VMEM((WELT,), jnp.int32),
            pltpu.VMEM((CK, W128), F32),
            pltpu.VMEM((CK, W128), F32),
            pltpu.VMEM((CK, W128), F32),
            pltpu.VMEM((CK, W128), F32),
            pltpu.SemaphoreType.DMA,
            pltpu.SemaphoreType.DMA,
            pltpu.SemaphoreType.DMA,
            pltpu.SemaphoreType.DMA,
            pltpu.SemaphoreType.DMA,
            pltpu.SemaphoreType.DMA,
            pltpu.SemaphoreType.DMA,
            pltpu.SemaphoreType.DMA,
        ],
    )
    def k4(a_hbm, msp_hbm, hits_hbm, sps_hbm, ga_hbm, gm_hbm,
           gh_v, gs_v, ra0, ra1, rm0, rm1,
           sga0, sga1, sgm0, sgm1, swa0, swa1, swm0, swm1):
        cid = lax.axis_index("c")
        sid = lax.axis_index("s")
        w = sid * 2 + cid
        eb = w * WELT

        pltpu.sync_copy(hits_hbm.at[pl.ds(eb, WELT)], gh_v)
        pltpu.sync_copy(sps_hbm.at[pl.ds(eb, WELT)], gs_v)

        def xform(r, _):
            pnum = (eb + r * 16) // E_PAD
            h16 = gh_v[pl.ds(r * 16, 16)]
            s16 = gs_v[pl.ds(r * 16, 16)]
            gh_v[pl.ds(r * 16, 16)] = jnp.minimum(h16, N - 1) + pnum * N
            gs_v[pl.ds(r * 16, 16)] = jnp.minimum(s16, N - 1)
            return 0

        lax.fori_loop(0, WELT // 16, xform, 0)

        def pair(kk, _):
            a = 2 * kk
            b = a + 1
            offa = eb + a * CK
            offb = eb + b * CK
            ca0 = pltpu.async_copy(
                a_hbm.at[gh_v.at[pl.ds(a * CK, CK)]], ra0, sga0)
            cm0 = pltpu.async_copy(
                msp_hbm.at[gs_v.at[pl.ds(a * CK, CK)]], rm0, sgm0)
            ca1 = pltpu.async_copy(
                a_hbm.at[gh_v.at[pl.ds(b * CK, CK)]], ra1, sga1)
            cm1 = pltpu.async_copy(
                msp_hbm.at[gs_v.at[pl.ds(b * CK, CK)]], rm1, sgm1)
            ca0.wait()
            cm0.wait()
            wa0 = pltpu.async_copy(ra0, ga_hbm.at[pl.ds(offa, CK)], swa0)
            wm0 = pltpu.async_copy(rm0, gm_hbm.at[pl.ds(offa, CK)], swm0)
            ca1.wait()
            cm1.wait()
            wa1 = pltpu.async_copy(ra1, ga_hbm.at[pl.ds(offb, CK)], swa1)
            wm1 = pltpu.async_copy(rm1, gm_hbm.at[pl.ds(offb, CK)], swm1)
            wa0.wait()
            wm0.wait()
            wa1.wait()
            wm1.wait()
            return 0

        lax.fori_loop(0, WCH // 2, pair, 0)

        t = WCH - 1
        offt = eb + t * CK
        ca0 = pltpu.async_copy(
            a_hbm.at[gh_v.at[pl.ds(t * CK, CK)]], ra0, sga0)
        cm0 = pltpu.async_copy(
            msp_hbm.at[gs_v.at[pl.ds(t * CK, CK)]], rm0, sgm0)
        ca0.wait()
        cm0.wait()
        pltpu.sync_copy(ra0, ga_hbm.at[pl.ds(offt, CK)])
        pltpu.sync_copy(rm0, gm_hbm.at[pl.ds(offt, CK)])

    return k4(a_flat, msp_tab, hits_flat, sps_flat)


def _k6(msg_flat, hits_flat, zeros_hbm):
    @functools.partial(
        pl.kernel,
        out_type=jax.ShapeDtypeStruct((P * N_PAD, W128), F32),
        mesh=_sc_mesh(),
        scratch_types=[
            pltpu.VMEM((CK,), jnp.int32),
            pltpu.VMEM((CK,), jnp.int32),
            pltpu.VMEM((CK,), jnp.int32),
            pltpu.VMEM((CK, W128), F32),
            pltpu.VMEM_SHARED((QACC, W128), F32),
            pltpu.SemaphoreType.DMA,
        ],
    )
    def k6(msg_hbm, hits_hbm, zeros_ref, ss_hbm,
           hit_v, eidx_v, sidx_v, rows_v, acc, sem):
        cid = lax.axis_index("c")
        sid = lax.axis_index("s")
        lane = jnp.arange(16, dtype=jnp.int32)

        for qq in range(2):
            q = cid * 2 + qq
            base = q * QR
            for p in range(P):
                _zero_acc(zeros_ref, acc, sid)
                plsc.subcore_barrier()

                def ebody(k, _):
                    j = sid + 16 * k
                    off = p * E_PAD + j * CK
                    pltpu.sync_copy(hits_hbm.at[pl.ds(off, CK)], hit_v)
                    for r in range(CK // 16):
                        h16 = hit_v[pl.ds(r * 16, 16)]
                        roff = r * 16 + lane
                        l16 = h16 - base
                        ok = (l16 >= 0) & (l16 < QR)
                        sidx_v[pl.ds(r * 16, 16)] = jnp.where(
                            ok, l16, TRASHQ + roff)
                        e16 = j * CK + roff
                        eidx_v[pl.ds(r * 16, 16)] = (
                            jnp.where(ok, e16, roff) + p * E_PAD)
                    pltpu.async_copy(msg_hbm.at[eidx_v], rows_v, sem).wait()
                    pltpu.sync_copy(rows_v, acc.at[sidx_v], add=True)
                    return 0

                lax.fori_loop(0, PER_T, ebody, 0)
                plsc.subcore_barrier()
                _copy_out(acc, ss_hbm, sid, q, p * N_PAD + base)
                plsc.subcore_barrier()

    return k6(msg_flat, hits_flat, zeros_hbm)


def kernel(m_u, m_v, m_y, edge_index_u, edge_index_v, edge_index_y,
           W3a, b3a, W3b, b3b, We1, be1, We2, be2, Wn1, bn1, Wn2, bn2):
    ms = jnp.stack([m_u, m_v, m_y])
    hits = jnp.stack([edge_index_u[0], edge_index_v[0], edge_index_y[0]])
    sps = jnp.stack([edge_index_u[1], edge_index_v[1], edge_index_y[1]])
    hits = hits.astype(jnp.int32)
    sps = sps.astype(jnp.int32)
    pad = jnp.full((P, E_PAD - E), N, dtype=jnp.int32)
    hits_flat = jnp.concatenate([hits, pad], axis=1).reshape(P * E_PAD)
    sps_flat = jnp.concatenate([sps, pad], axis=1).reshape(P * E_PAD)
    zeros_hbm = jnp.zeros((ZQ, W128), dtype=F32)

    w3a_pl = W3a.reshape(C, P, NF, SF).transpose(1, 0, 2, 3)
    we1_top = We1[:, :, :NF, :]
    we1_bot = We1[:, :, NF:, :]
    wk1 = jnp.concatenate([w3a_pl, we1_top], axis=-1)
    b3a_flat = b3a.reshape(1, FW)
    b3b_flat = b3b.reshape(1, FW)
    be1_flat = be1.reshape(P, 1, FW)
    w2bd = (We2[:, :, :, 0][:, :, :, None]
            * jnp.eye(C, dtype=F32)[None, :, None, :]).reshape(P, FW, C)
    be2_flat = be2.reshape(P, 1, C)
    ebd = jnp.repeat(jnp.eye(C, dtype=F32), SF, axis=1)

    t_all, a_all = _k1(ms, wk1)

    z = _k2(t_all.reshape(P * N, W128), hits_flat, sps_flat, zeros_hbm)

    msp_tab = _k3(z, b3a_flat, W3b, b3b_flat)

    ga, gm = _k4(a_all.reshape(P * N, W128), msp_tab, hits_flat, sps_flat)

    msg = _k5(ga.reshape(P, E_PAD, W128), gm.reshape(P, E_PAD, W128),
              we1_bot, be1_flat, w2bd, be2_flat, ebd)

    ssum = _k6(msg.reshape(P * E_PAD, W128), hits_flat,
               zeros_hbm).reshape(P, N_PAD, W128)

    return _k7(ms, ssum, Wn1, bn1, Wn2, bn2)
